# baseline (speedup 1.0000x reference)
import functools

import jax
import jax.numpy as jnp
from jax import lax
from jax.experimental import pallas as pl
from jax.experimental.pallas import tpu as pltpu
from jax.experimental.pallas import tpu_sc as plsc

_B = 4096
_H = 50
_D = 32
_DEMO = 10
_HID = 64

_NC, _NS = 2, 16
_NW = _NC * _NS
_SPW = _B // _NW
_SPC = 2
_CHUNKS = _SPW // _SPC
_GIDX = _SPC * _H
_GPAD = 104
_NBUF = 4


def _sc_gather_pool(ids_p, table):
    mesh = plsc.VectorSubcoreMesh(core_axis_name="c", subcore_axis_name="s")

    @functools.partial(
        pl.kernel,
        mesh=mesh,
        out_type=jax.ShapeDtypeStruct((_B, _D), jnp.float32),
        scratch_types=[
            pltpu.VMEM((_CHUNKS, _GPAD), jnp.int32),
            pltpu.VMEM((_NBUF * _GPAD, _D), jnp.float32),
            pltpu.VMEM((_SPW, _D), jnp.float32),
        ]
        + [pltpu.SemaphoreType.DMA] * _NBUF,
        compiler_params=pltpu.CompilerParams(use_tc_tiling_on_sc=False),
    )
    def k(ids_hbm, table_hbm, out_hbm, idx_v, rows_v, acc_v, *sems):
        wid = lax.axis_index("s") * _NC + lax.axis_index("c")
        pltpu.sync_copy(ids_hbm.at[wid], idx_v)

        def buf(b):
            return rows_v.at[pl.ds(b * _GPAD, _GPAD)]

        def fire(c, b):
            pltpu.async_copy(table_hbm.at[idx_v.at[c]], buf(b), sems[b])

        for b in range(_NBUF):
            fire(b, b)

        def rsum(bf, base, lo, hi, l):
            if hi - lo == 1:
                return bf[base + lo, pl.ds(16 * l, 16)]
            mid = (lo + hi) // 2
            return rsum(bf, base, lo, mid, l) + rsum(bf, base, mid, hi, l)

        @pl.loop(0, _CHUNKS, step=_NBUF)
        def _(j):
            for b in range(_NBUF):
                c = j + b
                pltpu.make_async_copy(
                    table_hbm.at[idx_v.at[c]], buf(b), sems[b]).wait()
                bf = buf(b)
                for s in range(_SPC):
                    for l in range(_D // 16):
                        acc_v[c * _SPC + s, pl.ds(16 * l, 16)] = rsum(
                            bf, s * _H, 0, _H, l)

                @pl.when(c + _NBUF < _CHUNKS)
                def _():
                    fire(c + _NBUF, b)

        pltpu.sync_copy(acc_v, out_hbm.at[pl.ds(wid * _SPW, _SPW)])

    return k(ids_p, table)


def _tc_mlp(pooled, demo, w1, b1, w2, b2):
    def body(mv_ref, demo_ref, w1_ref, b1_ref, w2_ref, b2_ref, out_ref):
        mv = mv_ref[...] * (1.0 / _H)
        w1m = w1_ref[...]
        x1 = lax.dot_general(mv, w1m[:, :_D], (((1,), (1,)), ((), ())),
                             preferred_element_type=jnp.float32)
        x2 = lax.dot_general(demo_ref[...], w1m[:, _D:],
                             (((1,), (1,)), ((), ())),
                             preferred_element_type=jnp.float32)
        h = jax.nn.relu(x1 + x2 + b1_ref[...])
        o = jnp.sum(h * w2_ref[...], axis=1, keepdims=True) + b2_ref[0]
        out_ref[...] = jax.nn.sigmoid(o)

    return pl.pallas_call(
        body,
        in_specs=[pl.BlockSpec(memory_space=pltpu.VMEM)] * 5
        + [pl.BlockSpec(memory_space=pltpu.SMEM)],
        out_specs=pl.BlockSpec(memory_space=pltpu.VMEM),
        out_shape=jax.ShapeDtypeStruct((_B, 1), jnp.float32),
    )(pooled, demo, w1, b1.reshape(1, _HID), w2, b2)


def kernel(med_ids, demo_features, embed_table, W1, b1, W2, b2):
    ids = med_ids.astype(jnp.int32).reshape(_NW, _CHUNKS, _GIDX)
    ids_p = jnp.pad(ids, ((0, 0), (0, 0), (0, _GPAD - _GIDX)))
    pooled = _sc_gather_pool(ids_p, embed_table)
    return _tc_mlp(pooled, demo_features, W1, b1, W2, b2)


# speedup vs baseline: 1.2041x; 1.2041x over previous
import functools

import jax
import jax.numpy as jnp
from jax import lax
from jax.experimental import pallas as pl
from jax.experimental.pallas import tpu as pltpu
from jax.experimental.pallas import tpu_sc as plsc

_B = 4096
_H = 50
_D = 32
_DEMO = 10
_HID = 64

_NC, _NS = 2, 16
_NW = _NC * _NS
_SPW = _B // _NW
_SPC = 2
_CHUNKS = _SPW // _SPC
_GIDX = _SPC * _H
_GPAD = 104
_NBUF = 4


def _sc_gather_pool(ids_p, table_bf):
    mesh = plsc.VectorSubcoreMesh(core_axis_name="c", subcore_axis_name="s")

    @functools.partial(
        pl.kernel,
        mesh=mesh,
        out_type=jax.ShapeDtypeStruct((_B, _D), jnp.float32),
        scratch_types=[
            pltpu.VMEM((_CHUNKS, _GPAD), jnp.int32),
            pltpu.VMEM((_NBUF * _GPAD, _D), jnp.bfloat16),
            pltpu.VMEM((_SPW, _D), jnp.float32),
        ]
        + [pltpu.SemaphoreType.DMA] * _NBUF,
        compiler_params=pltpu.CompilerParams(
            use_tc_tiling_on_sc=False, needs_layout_passes=False),
    )
    def k(ids_hbm, table_hbm, out_hbm, idx_v, rows_v, acc_v, *sems):
        wid = lax.axis_index("s") * _NC + lax.axis_index("c")
        pltpu.sync_copy(ids_hbm.at[wid], idx_v)

        def buf(b):
            return rows_v.at[pl.ds(b * _GPAD, _GPAD)]

        def fire(c, b):
            pltpu.async_copy(table_hbm.at[idx_v.at[c]], buf(b), sems[b])

        for b in range(_NBUF):
            fire(b, b)

        hi_mask = jnp.full((16,), 0xFFFF0000, dtype=jnp.uint32)

        def load_row(bf, r):
            xi = plsc.bitcast(bf[r, :], jnp.uint32)
            ev = plsc.bitcast(xi << 16, jnp.float32)
            od = plsc.bitcast(xi & hi_mask, jnp.float32)
            return ev, od

        def rsum(bf, base, lo, hi):
            if hi - lo == 1:
                return load_row(bf, base + lo)
            mid = (lo + hi) // 2
            a = rsum(bf, base, lo, mid)
            b = rsum(bf, base, mid, hi)
            return a[0] + b[0], a[1] + b[1]

        @pl.loop(0, _CHUNKS, step=_NBUF)
        def _(j):
            for b in range(_NBUF):
                c = j + b
                pltpu.make_async_copy(
                    table_hbm.at[idx_v.at[c]], buf(b), sems[b]).wait()
                bf = buf(b)
                for s in range(_SPC):
                    ev, od = rsum(bf, s * _H, 0, _H)
                    acc_v[c * _SPC + s, pl.ds(0, 16)] = ev
                    acc_v[c * _SPC + s, pl.ds(16, 16)] = od

                @pl.when(c + _NBUF < _CHUNKS)
                def _():
                    fire(c + _NBUF, b)

        pltpu.sync_copy(acc_v, out_hbm.at[pl.ds(wid * _SPW, _SPW)])

    return k(ids_p, table_bf)


def _tc_mlp(pooled, demo, w1, b1, w2, b2):
    def body(mv_ref, demo_ref, w1_ref, b1_ref, w2_ref, b2_ref, out_ref):
        mv = mv_ref[...] * (1.0 / _H)
        w1m = w1_ref[...]
        x1 = lax.dot_general(mv, w1m[:, :_D], (((1,), (1,)), ((), ())),
                             preferred_element_type=jnp.float32)
        x2 = lax.dot_general(demo_ref[...], w1m[:, _D:],
                             (((1,), (1,)), ((), ())),
                             preferred_element_type=jnp.float32)
        h = jax.nn.relu(x1 + x2 + b1_ref[...])
        o = jnp.sum(h * w2_ref[...], axis=1, keepdims=True) + b2_ref[0]
        out_ref[...] = jax.nn.sigmoid(o)

    return pl.pallas_call(
        body,
        in_specs=[pl.BlockSpec(memory_space=pltpu.VMEM)] * 5
        + [pl.BlockSpec(memory_space=pltpu.SMEM)],
        out_specs=pl.BlockSpec(memory_space=pltpu.VMEM),
        out_shape=jax.ShapeDtypeStruct((_B, 1), jnp.float32),
    )(pooled, demo, w1, b1.reshape(1, _HID), w2, b2)


def kernel(med_ids, demo_features, embed_table, W1, b1, W2, b2):
    ids = med_ids.astype(jnp.int32).reshape(_NW, _CHUNKS, _GIDX)
    ids_p = jnp.pad(ids, ((0, 0), (0, 0), (0, _GPAD - _GIDX)))
    pooled = _sc_gather_pool(ids_p, embed_table.astype(jnp.bfloat16))
    perm = list(range(0, _D, 2)) + list(range(1, _D, 2))
    w1p = jnp.concatenate([W1[:, :_D][:, perm], W1[:, _D:]], axis=1)
    return _tc_mlp(pooled, demo_features, w1p, b1, W2, b2)


# speedup vs baseline: 1.5208x; 1.2631x over previous
import functools

import jax
import jax.numpy as jnp
from jax import lax
from jax.experimental import pallas as pl
from jax.experimental.pallas import tpu as pltpu
from jax.experimental.pallas import tpu_sc as plsc

_V = 100000
_B = 4096
_H = 50
_D = 32
_DEMO = 10
_HID = 64

_NC, _NS = 2, 16
_NW = _NC * _NS
_SPW = _B // _NW
_SPC = 2
_CHUNKS = _SPW // _SPC
_GIDX = _SPC * _H
_GPAD = 104
_NBUF = 4


def _sc_gather_pool(ids_p, table_bf):
    mesh = plsc.VectorSubcoreMesh(core_axis_name="c", subcore_axis_name="s")

    @functools.partial(
        pl.kernel,
        mesh=mesh,
        out_type=jax.ShapeDtypeStruct((_B, _D), jnp.float32),
        scratch_types=[
            pltpu.VMEM((_CHUNKS, _GPAD), jnp.int32),
            pltpu.VMEM((_NBUF * _GPAD, _D), jnp.bfloat16),
            pltpu.VMEM((_SPW, _D), jnp.float32),
            pltpu.VMEM_SHARED((_V, _D), jnp.bfloat16),
        ]
        + [pltpu.SemaphoreType.DMA] * _NBUF,
        compiler_params=pltpu.CompilerParams(
            use_tc_tiling_on_sc=False, needs_layout_passes=False),
    )
    def k(ids_hbm, table_hbm, out_hbm, idx_v, rows_v, acc_v, tab_sh, *sems):
        wid = lax.axis_index("s") * _NC + lax.axis_index("c")
        sid = lax.axis_index("s")
        rows_per_tile = _V // _NS
        pltpu.sync_copy(table_hbm.at[pl.ds(sid * rows_per_tile, rows_per_tile)],
                        tab_sh.at[pl.ds(sid * rows_per_tile, rows_per_tile)])
        pltpu.sync_copy(ids_hbm.at[wid], idx_v)
        plsc.subcore_barrier()

        def buf(b):
            return rows_v.at[pl.ds(b * _GPAD, _GPAD)]

        def fire(c, b):
            pltpu.async_copy(tab_sh.at[idx_v.at[c]], buf(b), sems[b])

        for b in range(_NBUF):
            fire(b, b)

        hi_mask = jnp.full((16,), 0xFFFF0000, dtype=jnp.uint32)

        def load_row(bf, r):
            xi = plsc.bitcast(bf[r, :], jnp.uint32)
            ev = plsc.bitcast(xi << 16, jnp.float32)
            od = plsc.bitcast(xi & hi_mask, jnp.float32)
            return ev, od

        def rsum(bf, base, lo, hi):
            if hi - lo == 1:
                return load_row(bf, base + lo)
            mid = (lo + hi) // 2
            a = rsum(bf, base, lo, mid)
            b = rsum(bf, base, mid, hi)
            return a[0] + b[0], a[1] + b[1]

        @pl.loop(0, _CHUNKS, step=_NBUF)
        def _(j):
            for b in range(_NBUF):
                c = j + b
                pltpu.make_async_copy(
                    tab_sh.at[idx_v.at[c]], buf(b), sems[b]).wait()
                bf = buf(b)
                for s in range(_SPC):
                    ev, od = rsum(bf, s * _H, 0, _H)
                    acc_v[c * _SPC + s, pl.ds(0, 16)] = ev
                    acc_v[c * _SPC + s, pl.ds(16, 16)] = od

                @pl.when(c + _NBUF < _CHUNKS)
                def _():
                    fire(c + _NBUF, b)

        pltpu.sync_copy(acc_v, out_hbm.at[pl.ds(wid * _SPW, _SPW)])

    return k(ids_p, table_bf)


def _tc_mlp(pooled, demo, w1, b1, w2, b2):
    def body(mv_ref, demo_ref, w1_ref, b1_ref, w2_ref, b2_ref, out_ref):
        mv = mv_ref[...] * (1.0 / _H)
        w1m = w1_ref[...]
        x1 = lax.dot_general(mv, w1m[:, :_D], (((1,), (1,)), ((), ())),
                             preferred_element_type=jnp.float32)
        x2 = lax.dot_general(demo_ref[...], w1m[:, _D:],
                             (((1,), (1,)), ((), ())),
                             preferred_element_type=jnp.float32)
        h = jax.nn.relu(x1 + x2 + b1_ref[...])
        o = jnp.sum(h * w2_ref[...], axis=1, keepdims=True) + b2_ref[0]
        out_ref[...] = jax.nn.sigmoid(o)

    return pl.pallas_call(
        body,
        in_specs=[pl.BlockSpec(memory_space=pltpu.VMEM)] * 5
        + [pl.BlockSpec(memory_space=pltpu.SMEM)],
        out_specs=pl.BlockSpec(memory_space=pltpu.VMEM),
        out_shape=jax.ShapeDtypeStruct((_B, 1), jnp.float32),
    )(pooled, demo, w1, b1.reshape(1, _HID), w2, b2)


def kernel(med_ids, demo_features, embed_table, W1, b1, W2, b2):
    ids = med_ids.astype(jnp.int32).reshape(_NW, _CHUNKS, _GIDX)
    ids_p = jnp.pad(ids, ((0, 0), (0, 0), (0, _GPAD - _GIDX)))
    pooled = _sc_gather_pool(ids_p, embed_table.astype(jnp.bfloat16))
    perm = list(range(0, _D, 2)) + list(range(1, _D, 2))
    w1p = jnp.concatenate([W1[:, :_D][:, perm], W1[:, _D:]], axis=1)
    return _tc_mlp(pooled, demo_features, w1p, b1, W2, b2)
